# Initial kernel scaffold; baseline (speedup 1.0000x reference)
import jax
import jax.numpy as jnp
from jax.experimental import pallas as pl


def kernel(x, edge_index, batch, W1, b1, W2, b2, W3, b3, Wlin, blin):
    raise NotImplementedError("write your pallas kernel here")


# speedup vs baseline: 3.9829x; 3.9829x over previous
import functools

import jax
import jax.numpy as jnp
from jax import lax
from jax.experimental import pallas as pl
from jax.experimental.pallas import tpu as pltpu
from jax.experimental.pallas import tpu_sc as plsc

_N = 10000
_E = 160000
_D = 256
_H = 512
_C = 16
_G = 64

_NP = 10240
_EP = 163840
_NCH = 4
_CHW = 128
_NB = 1024
_NRB = _NP // _NB

_NSUB = 16
_KE = 128
_EPT = _EP // _NSUB
_NBLK = _EPT // _KE
_EPT2 = _EP // (2 * _NSUB)
_NBLK2 = _EPT2 // _KE
_RPT = _NP // _NSUB
_ZR = 64

_sc_mesh = plsc.VectorSubcoreMesh(core_axis_name="c", subcore_axis_name="s")


@functools.partial(
    pl.kernel,
    mesh=_sc_mesh,
    out_type=jax.ShapeDtypeStruct((2, _NP, _CHW), jnp.float32),
    scratch_types=[
        pltpu.VMEM((_KE,), jnp.int32),
        pltpu.VMEM((_KE, _CHW), jnp.float32),
        pltpu.VMEM((_ZR, _CHW), jnp.float32),
        pltpu.VMEM_SHARED((_NP, _CHW), jnp.float32),
        pltpu.SemaphoreType.DMA,
    ],
)
def _deg_sc(dst_hbm, out_hbm, dst_v, ones_v, zro_v, acc_sh, sem):
    cid = lax.axis_index("c")
    sid = lax.axis_index("s")

    @pl.loop(0, _KE)
    def _(r):
        @pl.loop(0, _CHW, step=16)
        def _(l):
            ones_v[r, pl.ds(l, 16)] = jnp.ones((16,), jnp.float32)

    @pl.loop(0, _ZR)
    def _(r):
        @pl.loop(0, _CHW, step=16)
        def _(l):
            zro_v[r, pl.ds(l, 16)] = jnp.zeros((16,), jnp.float32)

    @pl.loop(0, _RPT, step=_ZR)
    def _(r0):
        pltpu.sync_copy(zro_v, acc_sh.at[pl.ds(sid * _RPT + r0, _ZR)])

    plsc.subcore_barrier()
    base = (sid * 2 + cid) * _EPT2

    @pl.loop(0, _NBLK2)
    def _(b):
        pltpu.sync_copy(dst_hbm.at[pl.ds(base + b * _KE, _KE)], dst_v)
        pltpu.sync_copy(ones_v, acc_sh.at[dst_v], add=True)

    plsc.subcore_barrier()

    @pl.loop(0, _RPT, step=_ZR)
    def _(r0):
        r = sid * _RPT + r0
        pltpu.sync_copy(acc_sh.at[pl.ds(r, _ZR)], out_hbm.at[cid, pl.ds(r, _ZR)])


@functools.partial(
    pl.kernel,
    mesh=_sc_mesh,
    out_type=jax.ShapeDtypeStruct((_NCH, _NP, _CHW), jnp.float32),
    scratch_types=[
        pltpu.VMEM((_KE,), jnp.int32),
        pltpu.VMEM((_KE,), jnp.int32),
        pltpu.VMEM((_KE, _CHW), jnp.float32),
        pltpu.VMEM((_ZR, _CHW), jnp.float32),
        pltpu.VMEM_SHARED((_NP, _CHW), jnp.float32),
        pltpu.SemaphoreType.DMA,
    ],
)
def _agg_sc(hp_hbm, srcc_hbm, dst_hbm, out_hbm, src_v, dst_v, rows_v, zro_v,
            acc_sh, sem):
    cid = lax.axis_index("c")
    sid = lax.axis_index("s")

    @pl.loop(0, _ZR)
    def _(r):
        @pl.loop(0, _CHW, step=16)
        def _(l):
            zro_v[r, pl.ds(l, 16)] = jnp.zeros((16,), jnp.float32)

    for j in range(2):
        chunk = 2 * cid + j

        @pl.loop(0, _RPT, step=_ZR)
        def _(r0):
            pltpu.sync_copy(zro_v, acc_sh.at[pl.ds(sid * _RPT + r0, _ZR)])

        plsc.subcore_barrier()
        base = sid * _EPT

        @pl.loop(0, _NBLK)
        def _(b):
            e0 = base + b * _KE
            pltpu.sync_copy(srcc_hbm.at[chunk, pl.ds(e0, _KE)], src_v)
            pltpu.sync_copy(dst_hbm.at[pl.ds(e0, _KE)], dst_v)
            pltpu.async_copy(hp_hbm.at[src_v], rows_v, sem).wait()
            pltpu.sync_copy(rows_v, acc_sh.at[dst_v], add=True)

        plsc.subcore_barrier()

        @pl.loop(0, _RPT, step=_ZR)
        def _(r0):
            r = sid * _RPT + r0
            pltpu.sync_copy(acc_sh.at[pl.ds(r, _ZR)],
                            out_hbm.at[chunk, pl.ds(r, _ZR)])

        plsc.subcore_barrier()


def _dinv_of(dg_ref):
    deg = dg_ref[0, :, 0:1] + dg_ref[1, :, 0:1] + 1.0
    return lax.rsqrt(deg)


def _mm1_body(x_ref, w_ref, dg_ref, o_ref):
    dinv = _dinv_of(dg_ref)
    h = jnp.dot(x_ref[...], w_ref[...], preferred_element_type=jnp.float32)
    o_ref[0] = h * dinv


_mm1 = pl.pallas_call(
    _mm1_body,
    grid=(_NRB, _NCH),
    in_specs=[
        pl.BlockSpec((_NB, _D), lambda i, c: (i, 0)),
        pl.BlockSpec((_D, _CHW), lambda i, c: (0, c)),
        pl.BlockSpec((2, _NB, _CHW), lambda i, c: (0, i, 0)),
    ],
    out_specs=pl.BlockSpec((1, _NB, _CHW), lambda i, c: (c, i, 0)),
    out_shape=jax.ShapeDtypeStruct((_NCH, _NP, _CHW), jnp.float32),
)


def _mmk_body(acc_ref, hp_ref, dg_ref, b_ref, w_ref, o_ref):
    dinv = _dinv_of(dg_ref)
    out = jnp.zeros((_NB, _CHW), jnp.float32)
    for kc in range(_NCH):
        xk = jnp.maximum(
            dinv * (acc_ref[kc] + hp_ref[kc]) + b_ref[kc][None, :], 0.0)
        out = out + jnp.dot(xk, w_ref[pl.ds(kc * _CHW, _CHW), :],
                            preferred_element_type=jnp.float32)
    o_ref[0] = out * dinv


_mmk = pl.pallas_call(
    _mmk_body,
    grid=(_NRB, _NCH),
    in_specs=[
        pl.BlockSpec((_NCH, _NB, _CHW), lambda i, c: (0, i, 0)),
        pl.BlockSpec((_NCH, _NB, _CHW), lambda i, c: (0, i, 0)),
        pl.BlockSpec((2, _NB, _CHW), lambda i, c: (0, i, 0)),
        pl.BlockSpec((_NCH, _CHW), lambda i, c: (0, 0)),
        pl.BlockSpec((_H, _CHW), lambda i, c: (0, c)),
    ],
    out_specs=pl.BlockSpec((1, _NB, _CHW), lambda i, c: (c, i, 0)),
    out_shape=jax.ShapeDtypeStruct((_NCH, _NP, _CHW), jnp.float32),
)


def _fin_body(acc_ref, hp_ref, dg_ref, b_ref, bt_ref, wl_ref, bl_ref, o_ref,
              sum_s, cnt_s):
    i = pl.program_id(0)

    @pl.when(i == 0)
    def _():
        sum_s[...] = jnp.zeros_like(sum_s)
        cnt_s[...] = jnp.zeros_like(cnt_s)

    dinv = _dinv_of(dg_ref)
    z = jnp.zeros((_NB, _C), jnp.float32)
    for kc in range(_NCH):
        ok = dinv * (acc_ref[kc] + hp_ref[kc]) + b_ref[kc][None, :]
        z = z + jnp.dot(ok, wl_ref[pl.ds(kc * _CHW, _CHW), :],
                        preferred_element_type=jnp.float32)
    pt = (bt_ref[...] == lax.broadcasted_iota(jnp.int32, (_NB, _G), 1))
    pt = pt.astype(jnp.float32)
    dn = (((0,), (0,)), ((), ()))
    sum_s[...] += lax.dot_general(pt, z, dn,
                                  preferred_element_type=jnp.float32)
    cnt_s[...] += lax.dot_general(pt, jnp.ones((_NB, _C), jnp.float32), dn,
                                  preferred_element_type=jnp.float32)

    @pl.when(i == _NRB - 1)
    def _():
        o_ref[...] = sum_s[...] / jnp.maximum(cnt_s[...], 1.0) + bl_ref[...]


_fin = pl.pallas_call(
    _fin_body,
    grid=(_NRB,),
    in_specs=[
        pl.BlockSpec((_NCH, _NB, _CHW), lambda i: (0, i, 0)),
        pl.BlockSpec((_NCH, _NB, _CHW), lambda i: (0, i, 0)),
        pl.BlockSpec((2, _NB, _CHW), lambda i: (0, i, 0)),
        pl.BlockSpec((_NCH, _CHW), lambda i: (0, 0)),
        pl.BlockSpec((_NB, 1), lambda i: (i, 0)),
        pl.BlockSpec((_H, _C), lambda i: (0, 0)),
        pl.BlockSpec((1, _C), lambda i: (0, 0)),
    ],
    out_specs=pl.BlockSpec((_G, _C), lambda i: (0, 0)),
    out_shape=jax.ShapeDtypeStruct((_G, _C), jnp.float32),
    scratch_shapes=[
        pltpu.VMEM((_G, _C), jnp.float32),
        pltpu.VMEM((_G, _C), jnp.float32),
    ],
)


def kernel(x, edge_index, batch, W1, b1, W2, b2, W3, b3, Wlin, blin):
    x_p = jnp.pad(x, ((0, _NP - _N), (0, 0)))
    src_p = jnp.pad(edge_index[0], (0, _EP - _E))
    dst_p = jnp.pad(edge_index[1], (0, _EP - _E), constant_values=_NP - 1)
    srcc = src_p[None, :] + (jnp.arange(_NCH, dtype=jnp.int32) * _NP)[:, None]
    batch_p = jnp.pad(batch, (0, _NP - _N), constant_values=_G)
    batch_p = batch_p.reshape(_NP, 1)

    degh = _deg_sc(dst_p)
    hp1 = _mm1(x_p, W1, degh)
    acc1 = _agg_sc(hp1.reshape(_NCH * _NP, _CHW), srcc, dst_p)
    hp2 = _mmk(acc1, hp1, degh, b1.reshape(_NCH, _CHW), W2)
    acc2 = _agg_sc(hp2.reshape(_NCH * _NP, _CHW), srcc, dst_p)
    hp3 = _mmk(acc2, hp2, degh, b2.reshape(_NCH, _CHW), W3)
    acc3 = _agg_sc(hp3.reshape(_NCH * _NP, _CHW), srcc, dst_p)
    return _fin(acc3, hp3, degh, b3.reshape(_NCH, _CHW), batch_p, Wlin,
                blin.reshape(1, _C))


# speedup vs baseline: 5.0157x; 1.2593x over previous
import functools

import jax
import jax.numpy as jnp
from jax import lax
from jax.experimental import pallas as pl
from jax.experimental.pallas import tpu as pltpu
from jax.experimental.pallas import tpu_sc as plsc

_N = 10000
_E = 160000
_D = 256
_H = 512
_C = 16
_G = 64

_NP = 10240
_EP = 163840
_NCH = 4
_CHW = 128
_NB = 1024
_NRB = _NP // _NB

_NSUB = 16
_KE = 128
_STG = 16
_EPT = _EP // _NSUB
_NBLK = _EPT // _KE
_EPT2 = _EP // (2 * _NSUB)
_NBLK2 = _EPT2 // _KE
_RPT = _NP // _NSUB
_ZR = 64

_sc_mesh = plsc.VectorSubcoreMesh(core_axis_name="c", subcore_axis_name="s")


@functools.partial(
    pl.kernel,
    mesh=_sc_mesh,
    out_type=jax.ShapeDtypeStruct((2, _NP, _CHW), jnp.float32),
    scratch_types=[
        pltpu.VMEM((_KE,), jnp.int32),
        pltpu.VMEM((_KE, _CHW), jnp.float32),
        pltpu.VMEM((_ZR, _CHW), jnp.float32),
        pltpu.VMEM_SHARED((_NP, _CHW), jnp.float32),
        pltpu.SemaphoreType.DMA,
    ],
)
def _deg_sc(dst_hbm, out_hbm, dst_v, ones_v, zro_v, acc_sh, sem):
    cid = lax.axis_index("c")
    sid = lax.axis_index("s")

    @pl.loop(0, _KE)
    def _(r):
        @pl.loop(0, _CHW, step=16)
        def _(l):
            ones_v[r, pl.ds(l, 16)] = jnp.ones((16,), jnp.float32)

    @pl.loop(0, _ZR)
    def _(r):
        @pl.loop(0, _CHW, step=16)
        def _(l):
            zro_v[r, pl.ds(l, 16)] = jnp.zeros((16,), jnp.float32)

    @pl.loop(0, _RPT, step=_ZR)
    def _(r0):
        pltpu.sync_copy(zro_v, acc_sh.at[pl.ds(sid * _RPT + r0, _ZR)])

    plsc.subcore_barrier()
    base = (sid * 2 + cid) * _EPT2

    @pl.loop(0, _NBLK2)
    def _(b):
        pltpu.sync_copy(dst_hbm.at[pl.ds(base + b * _KE, _KE)], dst_v)
        pltpu.sync_copy(ones_v, acc_sh.at[dst_v], add=True)

    plsc.subcore_barrier()

    @pl.loop(0, _RPT, step=_ZR)
    def _(r0):
        r = sid * _RPT + r0
        pltpu.sync_copy(acc_sh.at[pl.ds(r, _ZR)], out_hbm.at[cid, pl.ds(r, _ZR)])


@functools.partial(
    pl.kernel,
    mesh=_sc_mesh,
    out_type=jax.ShapeDtypeStruct((_NCH, _NP, _CHW), jnp.float32),
    scratch_types=[
        pltpu.VMEM((_STG, _KE), jnp.int32),
        pltpu.VMEM((_STG, _KE), jnp.int32),
        pltpu.VMEM((_KE, _CHW), jnp.float32),
        pltpu.VMEM((_KE, _CHW), jnp.float32),
        pltpu.VMEM((_ZR, _CHW), jnp.float32),
        pltpu.VMEM_SHARED((_NP, _CHW), jnp.float32),
        pltpu.SemaphoreType.DMA,
        pltpu.SemaphoreType.DMA,
    ],
)
def _agg_sc(hp_hbm, srcc_hbm, dst_hbm, out_hbm, srcb_v, dstb_v, rows0_v,
            rows1_v, zro_v, acc_sh, sem0, sem1):
    cid = lax.axis_index("c")
    sid = lax.axis_index("s")

    @pl.loop(0, _ZR)
    def _(r):
        @pl.loop(0, _CHW, step=16)
        def _(l):
            zro_v[r, pl.ds(l, 16)] = jnp.zeros((16,), jnp.float32)

    def _gather_start(buf, sem, blk):
        pltpu.async_copy(hp_hbm.at[srcb_v.at[blk]], buf, sem)

    def _gather_wait(buf, sem, blk):
        pltpu.make_async_copy(hp_hbm.at[srcb_v.at[blk]], buf, sem).wait()

    for j in range(2):
        chunk = 2 * cid + j

        @pl.loop(0, _RPT, step=_ZR)
        def _(r0):
            pltpu.sync_copy(zro_v, acc_sh.at[pl.ds(sid * _RPT + r0, _ZR)])

        plsc.subcore_barrier()

        @pl.loop(0, _NBLK // _STG)
        def _(s):
            pltpu.sync_copy(srcc_hbm.at[chunk, sid, pl.ds(s * _STG, _STG)],
                            srcb_v)
            pltpu.sync_copy(dst_hbm.at[sid, pl.ds(s * _STG, _STG)], dstb_v)
            _gather_start(rows0_v, sem0, 0)

            @pl.loop(0, _STG, step=2)
            def _(b):
                _gather_wait(rows0_v, sem0, b)
                _gather_start(rows1_v, sem1, b + 1)
                pltpu.sync_copy(rows0_v, acc_sh.at[dstb_v.at[b]], add=True)
                _gather_wait(rows1_v, sem1, b + 1)

                @pl.when(b + 2 < _STG)
                def _():
                    _gather_start(rows0_v, sem0, b + 2)

                pltpu.sync_copy(rows1_v, acc_sh.at[dstb_v.at[b + 1]],
                                add=True)

        plsc.subcore_barrier()

        @pl.loop(0, _RPT, step=_ZR)
        def _(r0):
            r = sid * _RPT + r0
            pltpu.sync_copy(acc_sh.at[pl.ds(r, _ZR)],
                            out_hbm.at[chunk, pl.ds(r, _ZR)])


def _dinv_of(dg_ref):
    deg = dg_ref[0, :, 0:1] + dg_ref[1, :, 0:1] + 1.0
    return lax.rsqrt(deg)


def _mm1_body(x_ref, w_ref, dg_ref, o_ref):
    dinv = _dinv_of(dg_ref)
    h = jnp.dot(x_ref[...], w_ref[...], preferred_element_type=jnp.float32)
    o_ref[0] = h * dinv


_mm1 = pl.pallas_call(
    _mm1_body,
    grid=(_NRB, _NCH),
    in_specs=[
        pl.BlockSpec((_NB, _D), lambda i, c: (i, 0)),
        pl.BlockSpec((_D, _CHW), lambda i, c: (0, c)),
        pl.BlockSpec((2, _NB, _CHW), lambda i, c: (0, i, 0)),
    ],
    out_specs=pl.BlockSpec((1, _NB, _CHW), lambda i, c: (c, i, 0)),
    out_shape=jax.ShapeDtypeStruct((_NCH, _NP, _CHW), jnp.float32),
)


def _mmk_body(acc_ref, hp_ref, dg_ref, b_ref, w_ref, o_ref):
    dinv = _dinv_of(dg_ref)
    out = jnp.zeros((_NB, _CHW), jnp.float32)
    for kc in range(_NCH):
        xk = jnp.maximum(
            dinv * (acc_ref[kc] + hp_ref[kc]) + b_ref[kc][None, :], 0.0)
        out = out + jnp.dot(xk, w_ref[pl.ds(kc * _CHW, _CHW), :],
                            preferred_element_type=jnp.float32)
    o_ref[0] = out * dinv


_mmk = pl.pallas_call(
    _mmk_body,
    grid=(_NRB, _NCH),
    in_specs=[
        pl.BlockSpec((_NCH, _NB, _CHW), lambda i, c: (0, i, 0)),
        pl.BlockSpec((_NCH, _NB, _CHW), lambda i, c: (0, i, 0)),
        pl.BlockSpec((2, _NB, _CHW), lambda i, c: (0, i, 0)),
        pl.BlockSpec((_NCH, _CHW), lambda i, c: (0, 0)),
        pl.BlockSpec((_H, _CHW), lambda i, c: (0, c)),
    ],
    out_specs=pl.BlockSpec((1, _NB, _CHW), lambda i, c: (c, i, 0)),
    out_shape=jax.ShapeDtypeStruct((_NCH, _NP, _CHW), jnp.float32),
)


def _fin_body(acc_ref, hp_ref, dg_ref, b_ref, bt_ref, wl_ref, bl_ref, o_ref,
              sum_s, cnt_s):
    i = pl.program_id(0)

    @pl.when(i == 0)
    def _():
        sum_s[...] = jnp.zeros_like(sum_s)
        cnt_s[...] = jnp.zeros_like(cnt_s)

    dinv = _dinv_of(dg_ref)
    z = jnp.zeros((_NB, _C), jnp.float32)
    for kc in range(_NCH):
        ok = dinv * (acc_ref[kc] + hp_ref[kc]) + b_ref[kc][None, :]
        z = z + jnp.dot(ok, wl_ref[pl.ds(kc * _CHW, _CHW), :],
                        preferred_element_type=jnp.float32)
    pt = (bt_ref[...] == lax.broadcasted_iota(jnp.int32, (_NB, _G), 1))
    pt = pt.astype(jnp.float32)
    dn = (((0,), (0,)), ((), ()))
    sum_s[...] += lax.dot_general(pt, z, dn,
                                  preferred_element_type=jnp.float32)
    cnt_s[...] += lax.dot_general(pt, jnp.ones((_NB, _C), jnp.float32), dn,
                                  preferred_element_type=jnp.float32)

    @pl.when(i == _NRB - 1)
    def _():
        o_ref[...] = sum_s[...] / jnp.maximum(cnt_s[...], 1.0) + bl_ref[...]


_fin = pl.pallas_call(
    _fin_body,
    grid=(_NRB,),
    in_specs=[
        pl.BlockSpec((_NCH, _NB, _CHW), lambda i: (0, i, 0)),
        pl.BlockSpec((_NCH, _NB, _CHW), lambda i: (0, i, 0)),
        pl.BlockSpec((2, _NB, _CHW), lambda i: (0, i, 0)),
        pl.BlockSpec((_NCH, _CHW), lambda i: (0, 0)),
        pl.BlockSpec((_NB, 1), lambda i: (i, 0)),
        pl.BlockSpec((_H, _C), lambda i: (0, 0)),
        pl.BlockSpec((1, _C), lambda i: (0, 0)),
    ],
    out_specs=pl.BlockSpec((_G, _C), lambda i: (0, 0)),
    out_shape=jax.ShapeDtypeStruct((_G, _C), jnp.float32),
    scratch_shapes=[
        pltpu.VMEM((_G, _C), jnp.float32),
        pltpu.VMEM((_G, _C), jnp.float32),
    ],
)


def kernel(x, edge_index, batch, W1, b1, W2, b2, W3, b3, Wlin, blin):
    x_p = jnp.pad(x, ((0, _NP - _N), (0, 0)))
    src_p = jnp.pad(edge_index[0], (0, _EP - _E))
    dst_p = jnp.pad(edge_index[1], (0, _EP - _E), constant_values=_NP - 1)
    srcc = src_p[None, :] + (jnp.arange(_NCH, dtype=jnp.int32) * _NP)[:, None]
    srcc = srcc.reshape(_NCH, _NSUB, _NBLK, _KE)
    dst4 = dst_p.reshape(_NSUB, _NBLK, _KE)
    batch_p = jnp.pad(batch, (0, _NP - _N), constant_values=_G)
    batch_p = batch_p.reshape(_NP, 1)

    degh = _deg_sc(dst_p)
    hp1 = _mm1(x_p, W1, degh)
    acc1 = _agg_sc(hp1.reshape(_NCH * _NP, _CHW), srcc, dst4)
    hp2 = _mmk(acc1, hp1, degh, b1.reshape(_NCH, _CHW), W2)
    acc2 = _agg_sc(hp2.reshape(_NCH * _NP, _CHW), srcc, dst4)
    hp3 = _mmk(acc2, hp2, degh, b2.reshape(_NCH, _CHW), W3)
    acc3 = _agg_sc(hp3.reshape(_NCH * _NP, _CHW), srcc, dst4)
    return _fin(acc3, hp3, degh, b3.reshape(_NCH, _CHW), batch_p, Wlin,
                blin.reshape(1, _C))


# speedup vs baseline: 6.4329x; 1.2826x over previous
import functools

import jax
import jax.numpy as jnp
from jax import lax
from jax.experimental import pallas as pl
from jax.experimental.pallas import tpu as pltpu
from jax.experimental.pallas import tpu_sc as plsc

_N = 10000
_E = 160000
_D = 256
_H = 512
_C = 16
_G = 64

_NP = 10240
_EP = 163840
_CHW = 128
_NB = 1024
_NRB = _NP // _NB

_NSUB = 16
_KE = 128
_STG = 16
_EPT = _EP // _NSUB
_NBLK = _EPT // _KE
_EPT2 = _EP // (2 * _NSUB)
_NBLK2 = _EPT2 // _KE
_RPT = _NP // _NSUB
_ZR = 64

_sc_mesh = plsc.VectorSubcoreMesh(core_axis_name="c", subcore_axis_name="s")


@functools.partial(
    pl.kernel,
    mesh=_sc_mesh,
    out_type=jax.ShapeDtypeStruct((2, _NP, _CHW), jnp.float32),
    scratch_types=[
        pltpu.VMEM((_KE,), jnp.int32),
        pltpu.VMEM((_KE, _CHW), jnp.float32),
        pltpu.VMEM((_ZR, _CHW), jnp.float32),
        pltpu.VMEM_SHARED((_NP, _CHW), jnp.float32),
        pltpu.SemaphoreType.DMA,
    ],
)
def _deg_sc(dst_hbm, out_hbm, dst_v, ones_v, zro_v, acc_sh, sem):
    cid = lax.axis_index("c")
    sid = lax.axis_index("s")

    @pl.loop(0, _KE)
    def _(r):
        @pl.loop(0, _CHW, step=16)
        def _(l):
            ones_v[r, pl.ds(l, 16)] = jnp.ones((16,), jnp.float32)

    @pl.loop(0, _ZR)
    def _(r):
        @pl.loop(0, _CHW, step=16)
        def _(l):
            zro_v[r, pl.ds(l, 16)] = jnp.zeros((16,), jnp.float32)

    @pl.loop(0, _RPT, step=_ZR)
    def _(r0):
        pltpu.sync_copy(zro_v, acc_sh.at[pl.ds(sid * _RPT + r0, _ZR)])

    plsc.subcore_barrier()
    base = (sid * 2 + cid) * _EPT2

    @pl.loop(0, _NBLK2)
    def _(b):
        pltpu.sync_copy(dst_hbm.at[pl.ds(base + b * _KE, _KE)], dst_v)
        pltpu.sync_copy(ones_v, acc_sh.at[dst_v], add=True)

    plsc.subcore_barrier()

    @pl.loop(0, _RPT, step=_ZR)
    def _(r0):
        r = sid * _RPT + r0
        pltpu.sync_copy(acc_sh.at[pl.ds(r, _ZR)], out_hbm.at[cid, pl.ds(r, _ZR)])


def _make_agg(nch):
    jpc = nch // 2

    @functools.partial(
        pl.kernel,
        mesh=_sc_mesh,
        out_type=jax.ShapeDtypeStruct((nch, _NP, _CHW), jnp.float32),
        scratch_types=[
            pltpu.VMEM((_STG, _KE), jnp.int32),
            pltpu.VMEM((_STG, _KE), jnp.int32),
            pltpu.VMEM((_KE, _CHW), jnp.float32),
            pltpu.VMEM((_KE, _CHW), jnp.float32),
            pltpu.VMEM((_ZR, _CHW), jnp.float32),
            pltpu.VMEM_SHARED((_NP, _CHW), jnp.float32),
            pltpu.SemaphoreType.DMA,
            pltpu.SemaphoreType.DMA,
        ],
    )
    def agg(xd_hbm, srcc_hbm, dst_hbm, out_hbm, srcb_v, dstb_v, rows0_v,
            rows1_v, zro_v, acc_sh, sem0, sem1):
        cid = lax.axis_index("c")
        sid = lax.axis_index("s")

        @pl.loop(0, _ZR)
        def _(r):
            @pl.loop(0, _CHW, step=16)
            def _(l):
                zro_v[r, pl.ds(l, 16)] = jnp.zeros((16,), jnp.float32)

        def _gather_start(buf, sem, blk):
            pltpu.async_copy(xd_hbm.at[srcb_v.at[blk]], buf, sem)

        def _gather_wait(buf, sem, blk):
            pltpu.make_async_copy(xd_hbm.at[srcb_v.at[blk]], buf, sem).wait()

        for j in range(jpc):
            chunk = jpc * cid + j

            @pl.loop(0, _RPT, step=_ZR)
            def _(r0):
                pltpu.sync_copy(zro_v, acc_sh.at[pl.ds(sid * _RPT + r0, _ZR)])

            plsc.subcore_barrier()

            @pl.loop(0, _NBLK // _STG)
            def _(s):
                pltpu.sync_copy(
                    srcc_hbm.at[chunk, sid, pl.ds(s * _STG, _STG)], srcb_v)
                pltpu.sync_copy(dst_hbm.at[sid, pl.ds(s * _STG, _STG)], dstb_v)
                _gather_start(rows0_v, sem0, 0)

                @pl.loop(0, _STG, step=2)
                def _(b):
                    _gather_wait(rows0_v, sem0, b)
                    _gather_start(rows1_v, sem1, b + 1)
                    pltpu.sync_copy(rows0_v, acc_sh.at[dstb_v.at[b]], add=True)
                    _gather_wait(rows1_v, sem1, b + 1)

                    @pl.when(b + 2 < _STG)
                    def _():
                        _gather_start(rows0_v, sem0, b + 2)

                    pltpu.sync_copy(rows1_v, acc_sh.at[dstb_v.at[b + 1]],
                                    add=True)

            plsc.subcore_barrier()

            @pl.loop(0, _RPT, step=_ZR)
            def _(r0):
                r = sid * _RPT + r0
                pltpu.sync_copy(acc_sh.at[pl.ds(r, _ZR)],
                                out_hbm.at[chunk, pl.ds(r, _ZR)])

    return agg


_agg2 = _make_agg(2)
_agg4 = _make_agg(4)


def _dinv_of(dg_ref):
    deg = dg_ref[0, :, 0:1] + dg_ref[1, :, 0:1] + 1.0
    return lax.rsqrt(deg)


def _scale1_body(x_ref, dg_ref, o_ref):
    o_ref[0] = x_ref[...] * _dinv_of(dg_ref)


_scale1 = pl.pallas_call(
    _scale1_body,
    grid=(_NRB, _D // _CHW),
    in_specs=[
        pl.BlockSpec((_NB, _CHW), lambda i, c: (i, c)),
        pl.BlockSpec((2, _NB, _CHW), lambda i, c: (0, i, 0)),
    ],
    out_specs=pl.BlockSpec((1, _NB, _CHW), lambda i, c: (c, i, 0)),
    out_shape=jax.ShapeDtypeStruct((_D // _CHW, _NP, _CHW), jnp.float32),
)


def _make_mm(nch_in):
    def body(u_ref, xd_ref, dg_ref, b_ref, w_ref, o_ref):
        dinv = _dinv_of(dg_ref)
        out = jnp.zeros((_NB, _CHW), jnp.float32)
        for kc in range(nch_in):
            t = dinv * (u_ref[kc] + xd_ref[kc])
            out = out + jnp.dot(t, w_ref[pl.ds(kc * _CHW, _CHW), :],
                                preferred_element_type=jnp.float32)
        o_ref[0] = dinv * jnp.maximum(out + b_ref[0, 0][None, :], 0.0)

    return pl.pallas_call(
        body,
        grid=(_NRB, _H // _CHW),
        in_specs=[
            pl.BlockSpec((nch_in, _NB, _CHW), lambda i, c: (0, i, 0)),
            pl.BlockSpec((nch_in, _NB, _CHW), lambda i, c: (0, i, 0)),
            pl.BlockSpec((2, _NB, _CHW), lambda i, c: (0, i, 0)),
            pl.BlockSpec((1, 1, _CHW), lambda i, c: (c, 0, 0)),
            pl.BlockSpec((nch_in * _CHW, _CHW), lambda i, c: (0, c)),
        ],
        out_specs=pl.BlockSpec((1, _NB, _CHW), lambda i, c: (c, i, 0)),
        out_shape=jax.ShapeDtypeStruct((_H // _CHW, _NP, _CHW), jnp.float32),
    )


_mmA = _make_mm(_D // _CHW)
_mmB = _make_mm(_H // _CHW)


def _fin_body(u_ref, xd_ref, dg_ref, b3_ref, w3_ref, bt_ref, wl_ref, bl_ref,
              o_ref, sum_s, cnt_s):
    i = pl.program_id(0)

    @pl.when(i == 0)
    def _():
        sum_s[...] = jnp.zeros_like(sum_s)
        cnt_s[...] = jnp.zeros_like(cnt_s)

    dinv = _dinv_of(dg_ref)
    out3 = jnp.zeros((_NB, _H), jnp.float32)
    for kc in range(_H // _CHW):
        t = dinv * (u_ref[kc] + xd_ref[kc])
        out3 = out3 + jnp.dot(t, w3_ref[pl.ds(kc * _CHW, _CHW), :],
                              preferred_element_type=jnp.float32)
    out3 = out3 + b3_ref[...]
    z = jnp.dot(out3, wl_ref[...], preferred_element_type=jnp.float32)
    pt = (bt_ref[...] == lax.broadcasted_iota(jnp.int32, (_NB, _G), 1))
    pt = pt.astype(jnp.float32)
    dn = (((0,), (0,)), ((), ()))
    sum_s[...] += lax.dot_general(pt, z, dn,
                                  preferred_element_type=jnp.float32)
    cnt_s[...] += lax.dot_general(pt, jnp.ones((_NB, _C), jnp.float32), dn,
                                  preferred_element_type=jnp.float32)

    @pl.when(i == _NRB - 1)
    def _():
        o_ref[...] = sum_s[...] / jnp.maximum(cnt_s[...], 1.0) + bl_ref[...]


_fin = pl.pallas_call(
    _fin_body,
    grid=(_NRB,),
    in_specs=[
        pl.BlockSpec((_H // _CHW, _NB, _CHW), lambda i: (0, i, 0)),
        pl.BlockSpec((_H // _CHW, _NB, _CHW), lambda i: (0, i, 0)),
        pl.BlockSpec((2, _NB, _CHW), lambda i: (0, i, 0)),
        pl.BlockSpec((1, _H), lambda i: (0, 0)),
        pl.BlockSpec((_H, _H), lambda i: (0, 0)),
        pl.BlockSpec((_NB, 1), lambda i: (i, 0)),
        pl.BlockSpec((_H, _C), lambda i: (0, 0)),
        pl.BlockSpec((1, _C), lambda i: (0, 0)),
    ],
    out_specs=pl.BlockSpec((_G, _C), lambda i: (0, 0)),
    out_shape=jax.ShapeDtypeStruct((_G, _C), jnp.float32),
    scratch_shapes=[
        pltpu.VMEM((_G, _C), jnp.float32),
        pltpu.VMEM((_G, _C), jnp.float32),
    ],
)


def kernel(x, edge_index, batch, W1, b1, W2, b2, W3, b3, Wlin, blin):
    x_p = jnp.pad(x, ((0, _NP - _N), (0, 0)))
    src_p = jnp.pad(edge_index[0], (0, _EP - _E))
    dst_p = jnp.pad(edge_index[1], (0, _EP - _E), constant_values=_NP - 1)
    offs = jnp.arange(4, dtype=jnp.int32) * _NP
    srcc2 = (src_p[None, :] + offs[:2, None]).reshape(2, _NSUB, _NBLK, _KE)
    srcc4 = (src_p[None, :] + offs[:, None]).reshape(4, _NSUB, _NBLK, _KE)
    dst4 = dst_p.reshape(_NSUB, _NBLK, _KE)
    batch_p = jnp.pad(batch, (0, _NP - _N), constant_values=_G)
    batch_p = batch_p.reshape(_NP, 1)

    degh = _deg_sc(dst_p)
    xd = _scale1(x_p, degh)
    u1 = _agg2(xd.reshape(2 * _NP, _CHW), srcc2, dst4)
    x2d = _mmA(u1, xd, degh, b1.reshape(4, 1, _CHW), W1)
    u2 = _agg4(x2d.reshape(4 * _NP, _CHW), srcc4, dst4)
    x3d = _mmB(u2, x2d, degh, b2.reshape(4, 1, _CHW), W2)
    u3 = _agg4(x3d.reshape(4 * _NP, _CHW), srcc4, dst4)
    return _fin(u3, x3d, degh, b3.reshape(1, _H), W3, batch_p, Wlin,
                blin.reshape(1, _C))
